# baseline (speedup 1.0000x reference)
import functools

import jax
import jax.numpy as jnp
from jax import lax
from jax.experimental import pallas as pl
from jax.experimental.pallas import tpu as pltpu
from jax.experimental.pallas import tpu_sc as plsc

B = 16384
D = 16
NC, NS = 2, 16
NW = NC * NS
RPW = B // NW
BLK = 128
NBLK = RPW // BLK
L = 16


def _mf_body(user_hbm, item_hbm, uet_hbm, iet_hbm, ubt_hbm, ibt_hbm, out_hbm,
             uidx_v, iidx_v, ue_g, ie_g, ub_g, ib_g, out_v, sem):
    wid = lax.axis_index("s") * NC + lax.axis_index("c")

    pltpu.sync_copy(user_hbm.at[pl.ds(wid * NBLK, NBLK)], uidx_v)
    pltpu.sync_copy(item_hbm.at[pl.ds(wid * NBLK, NBLK)], iidx_v)

    copies = []
    for j in range(NBLK):
        uix = uidx_v.at[j]
        iix = iidx_v.at[j]
        for d in range(D):
            dst = pl.ds(d * RPW + j * BLK, BLK)
            copies.append(
                pltpu.async_copy(uet_hbm.at[d].at[uix], ue_g.at[dst], sem))
            copies.append(
                pltpu.async_copy(iet_hbm.at[d].at[iix], ie_g.at[dst], sem))
        bsl = pl.ds(j * BLK, BLK)
        copies.append(pltpu.async_copy(ubt_hbm.at[0].at[uix], ub_g.at[bsl], sem))
        copies.append(pltpu.async_copy(ibt_hbm.at[0].at[iix], ib_g.at[bsl], sem))
    for cp in copies:
        cp.wait()

    def chunk(t, _):
        base = t * L
        acc = ub_g[pl.ds(base, L)] + ib_g[pl.ds(base, L)]
        parts = []
        for d in range(D):
            u = ue_g[pl.ds(d * RPW + base, L)]
            it = ie_g[pl.ds(d * RPW + base, L)]
            parts.append(u * it)
        while len(parts) > 1:
            parts = [parts[k] + parts[k + 1] for k in range(0, len(parts), 2)]
        out_v[pl.ds(base, L)] = acc + parts[0]
        return _

    lax.fori_loop(0, RPW // L, chunk, None)

    pltpu.sync_copy(out_v, out_hbm.at[pl.ds(wid * RPW, RPW)])


@functools.partial(
    pl.kernel,
    out_type=jax.ShapeDtypeStruct((B,), jnp.float32),
    mesh=plsc.VectorSubcoreMesh(core_axis_name="c", subcore_axis_name="s"),
    compiler_params=pltpu.CompilerParams(
        needs_layout_passes=False, use_tc_tiling_on_sc=False),
    scratch_types=[
        pltpu.VMEM((NBLK, BLK), jnp.int32),
        pltpu.VMEM((NBLK, BLK), jnp.int32),
        pltpu.VMEM((D * RPW,), jnp.float32),
        pltpu.VMEM((D * RPW,), jnp.float32),
        pltpu.VMEM((RPW,), jnp.float32),
        pltpu.VMEM((RPW,), jnp.float32),
        pltpu.VMEM((RPW,), jnp.float32),
        pltpu.SemaphoreType.DMA,
    ],
)
def _mf_kernel(user2d, item2d, uet, iet, ubt, ibt, out,
               uidx_v, iidx_v, ue_g, ie_g, ub_g, ib_g, out_v, sem):
    _mf_body(user2d, item2d, uet, iet, ubt, ibt, out,
             uidx_v, iidx_v, ue_g, ie_g, ub_g, ib_g, out_v, sem)


def kernel(user, item, user_embeddings, item_embeddings, user_biases, item_biases):
    user2d = user.astype(jnp.int32).reshape(NW * NBLK, BLK)
    item2d = item.astype(jnp.int32).reshape(NW * NBLK, BLK)
    return _mf_kernel(user2d, item2d, user_embeddings.T, item_embeddings.T,
                      user_biases.T, item_biases.T)
